# baseline (speedup 1.0000x reference)
import functools

import jax
import jax.numpy as jnp
from jax import lax
from jax.experimental import pallas as pl
from jax.experimental.pallas import tpu as pltpu
from jax.experimental.pallas import tpu_sc as plsc

_N = 10000
_E = 320000
_EH = _E // 2
_H = 64
_W = 128
_STEPS = 4
_NC = 2
_NS = 16
_NW = _NC * _NS
_CH = 128


def _sc_mesh():
    return plsc.VectorSubcoreMesh(
        core_axis_name="c", subcore_axis_name="s",
        num_cores=_NC, num_subcores=_NS)


def _sc_gather_rows(table, idx, n_idx):
    per_w = n_idx // _NW
    n_full = per_w // _CH
    tail = per_w - n_full * _CH
    assert per_w % 8 == 0 and tail % 8 == 0 and n_full % 2 == 0

    @functools.partial(
        pl.kernel,
        out_type=jax.ShapeDtypeStruct((n_idx, _W), jnp.float32),
        mesh=_sc_mesh(),
        scratch_types=[
            pltpu.VMEM((per_w,), jnp.int32),
            pltpu.VMEM((2, _CH, _W), jnp.float32),
            pltpu.SemaphoreType.DMA,
            pltpu.SemaphoreType.DMA,
            pltpu.SemaphoreType.DMA,
            pltpu.SemaphoreType.DMA,
        ],
    )
    def k(table_hbm, idx_hbm, out_hbm, idx_v, rows_v, sg0, sg1, so0, so1):
        wid = lax.axis_index("s") * _NC + lax.axis_index("c")
        base = wid * per_w
        pltpu.sync_copy(idx_hbm.at[pl.ds(base, per_w)], idx_v)
        sg = (sg0, sg1)
        so = (so0, so1)

        def fire_g(i, s):
            pltpu.async_copy(
                table_hbm.at[idx_v.at[pl.ds(i * _CH, _CH)]], rows_v.at[s], sg[s])

        def wait_g(i, s):
            pltpu.make_async_copy(
                table_hbm.at[idx_v.at[pl.ds(i * _CH, _CH)]], rows_v.at[s], sg[s]).wait()

        def fire_o(i, s):
            pltpu.async_copy(
                rows_v.at[s], out_hbm.at[pl.ds(base + i * _CH, _CH)], so[s])

        def wait_o(i, s):
            pltpu.make_async_copy(
                rows_v.at[s], out_hbm.at[pl.ds(base + i * _CH, _CH)], so[s]).wait()

        fire_g(0, 0)

        def body(j, carry):
            for b in range(2):
                i = 2 * j + b
                wait_g(i, b)
                fire_o(i, b)

                @pl.when(i >= 1)
                def _():
                    wait_o(i - 1, 1 - b)

                @pl.when(i + 1 < n_full)
                def _():
                    fire_g(i + 1, 1 - b)
            return carry

        lax.fori_loop(0, n_full // 2, body, 0)
        wait_o(n_full - 1, (n_full - 1) % 2)
        if tail:
            t0 = n_full * _CH
            pltpu.async_copy(
                table_hbm.at[idx_v.at[pl.ds(t0, tail)]],
                rows_v.at[0, pl.ds(0, tail)], sg0).wait()
            pltpu.sync_copy(
                rows_v.at[0, pl.ds(0, tail)],
                out_hbm.at[pl.ds(base + t0, tail)])

    return k(table, idx)


def _sc_segment_partials(values, idx, zeros_nw):
    n_val = values.shape[0]
    per_w = n_val // _NW
    n_full = per_w // _CH
    tail = per_w - n_full * _CH
    assert per_w % 8 == 0 and tail % 8 == 0
    zr0 = 632
    zr_last = _N - zr0 * (_NS - 1)

    @functools.partial(
        pl.kernel,
        out_type=jax.ShapeDtypeStruct((2 * _N, _W), jnp.float32),
        mesh=_sc_mesh(),
        scratch_types=[
            pltpu.VMEM((2, _CH), jnp.int32),
            pltpu.VMEM((tail,), jnp.int32),
            pltpu.VMEM((2, _CH, _W), jnp.float32),
            pltpu.VMEM_SHARED((_N, _W), jnp.float32),
            pltpu.SemaphoreType.DMA,
            pltpu.SemaphoreType.DMA,
        ],
    )
    def k(val_hbm, idx_hbm, zero_hbm, out_hbm, idx_b, idx_t, rows_v, acc, sv0, sv1):
        cid = lax.axis_index("c")
        sid = lax.axis_index("s")
        wid = sid * _NC + cid
        base = wid * per_w

        @pl.when(sid < _NS - 1)
        def _():
            pltpu.sync_copy(zero_hbm.at[pl.ds(sid * zr0, zr0)],
                            acc.at[pl.ds(sid * zr0, zr0)])

        @pl.when(sid == _NS - 1)
        def _():
            pltpu.sync_copy(zero_hbm.at[pl.ds(sid * zr0, zr_last)],
                            acc.at[pl.ds(sid * zr0, zr_last)])

        plsc.subcore_barrier()
        sv = (sv0, sv1)

        def fire_v(i, s):
            pltpu.async_copy(
                val_hbm.at[pl.ds(base + i * _CH, _CH)], rows_v.at[s], sv[s])

        def wait_v(i, s):
            pltpu.make_async_copy(
                val_hbm.at[pl.ds(base + i * _CH, _CH)], rows_v.at[s], sv[s]).wait()

        pltpu.sync_copy(idx_hbm.at[pl.ds(base, _CH)], idx_b.at[0])
        fire_v(0, 0)

        def body(j, carry):
            for b in range(2):
                i = 2 * j + b

                @pl.when(i + 1 < n_full)
                def _():
                    pltpu.sync_copy(
                        idx_hbm.at[pl.ds(base + (i + 1) * _CH, _CH)],
                        idx_b.at[1 - b])
                    fire_v(i + 1, 1 - b)

                wait_v(i, b)
                pltpu.sync_copy(rows_v.at[b], acc.at[idx_b.at[b]], add=True)
            return carry

        lax.fori_loop(0, n_full // 2, body, 0)
        if n_full % 2:
            wait_v(n_full - 1, 0)
            pltpu.sync_copy(rows_v.at[0], acc.at[idx_b.at[0]], add=True)
        if tail:
            t0 = base + n_full * _CH
            pltpu.sync_copy(idx_hbm.at[pl.ds(t0, tail)], idx_t)
            pltpu.async_copy(
                val_hbm.at[pl.ds(t0, tail)],
                rows_v.at[0, pl.ds(0, tail)], sv0).wait()
            pltpu.sync_copy(rows_v.at[0, pl.ds(0, tail)], acc.at[idx_t], add=True)
        plsc.subcore_barrier()

        @pl.when(sid < _NS - 1)
        def _():
            pltpu.sync_copy(acc.at[pl.ds(sid * zr0, zr0)],
                            out_hbm.at[pl.ds(cid * _N + sid * zr0, zr0)])

        @pl.when(sid == _NS - 1)
        def _():
            pltpu.sync_copy(acc.at[pl.ds(sid * zr0, zr_last)],
                            out_hbm.at[pl.ds(cid * _N + sid * zr0, zr_last)])

    return k(values, idx, zeros_nw)


def _ln(h):
    o = jnp.full((_H, 1), 1.0 / _H, jnp.float32)
    mu = jnp.dot(h, o, preferred_element_type=jnp.float32)
    d = h - mu
    var = jnp.dot(d * d, o, preferred_element_type=jnp.float32)
    return d * lax.rsqrt(var + 1e-5)


def _ln_from_ext(hx):
    o = jnp.full((_H, 1), 1.0 / _H, jnp.float32)
    d = hx[:, 0:_H] - hx[:, _H:_H + 1]
    var = jnp.dot(d * d, o, preferred_element_type=jnp.float32)
    return d * lax.rsqrt(var + 1e-5)


def _dot(a, b):
    return jnp.dot(a, b, preferred_element_type=jnp.float32)


def _ext_w3(w3, b3):
    o = jnp.full((_H, 1), 1.0 / _H, jnp.float32)
    w3x = jnp.concatenate(
        [w3, w3 @ o, jnp.zeros((_H, _W - _H - 1), jnp.float32)], axis=1)
    b3r = b3.reshape(1, -1)
    b3x = jnp.concatenate(
        [b3r, b3r @ o, jnp.zeros((1, _W - _H - 1), jnp.float32)], axis=1)
    return w3x, b3x


def _rep(shape):
    return pl.BlockSpec(shape, lambda i: (0, 0))


_BN = 1000
_BEB = 1280


def _encoder_tc(pos12, radii, wb):
    (w1, b1), (w2, b2), (w3, b3) = wb
    grid = (_N // _BN,)

    def body(pos_ref, rad_ref, w1r, b1r, w2r, b2r, w3r, b3r, out_ref):
        p = pos_ref[...]
        r = rad_ref[...]
        vel = p[:, 2:12] - p[:, 0:10]
        cur = p[:, 10:12]
        inv_r = 1.0 / r
        nf = jnp.concatenate(
            [vel * inv_r,
             jnp.clip(jnp.concatenate([cur, 1.0 - cur], axis=1) * inv_r,
                      -4.0, 4.0),
             r], axis=1)
        h = jnp.maximum(_dot(nf, w1r[...]) + b1r[...], 0.0)
        h = jnp.maximum(_dot(h, w2r[...]) + b2r[...], 0.0)
        h = _dot(h, w3r[...]) + b3r[...]
        out_ref[:, 0:_H + 13] = jnp.concatenate([_ln(h), cur, vel, r], axis=1)

    return pl.pallas_call(
        body,
        grid=grid,
        in_specs=[
            pl.BlockSpec((_BN, 12), lambda i: (i, 0)),
            pl.BlockSpec((_BN, 1), lambda i: (i, 0)),
            _rep((15, _H)), _rep((1, _H)),
            _rep((_H, _H)), _rep((1, _H)),
            _rep((_H, _H)), _rep((1, _H)),
        ],
        out_specs=pl.BlockSpec((_BN, _W), lambda i: (i, 0)),
        out_shape=jax.ShapeDtypeStruct((_N, _W), jnp.float32),
    )(pos12, radii, w1, b1.reshape(1, -1), w2, b2.reshape(1, -1),
      w3, b3.reshape(1, -1))


def _edge_encoder_tc(g0, w1p, w1ds, b1, w2, b2, w3x, b3x):
    nb = _EH // _BEB

    def body(s_ref, r_ref, w1pr, w1dsr, b1r, w2r, b2r, w3r, b3r, out_ref):
        s = s_ref[...][:, 64:77]
        r = r_ref[...][:, 64:77]
        u = s - r
        rs = s[:, 12:13] + r[:, 12:13]
        inv = 1.0 / rs
        d2 = u[:, 0:1] * u[:, 0:1] + u[:, 1:2] * u[:, 1:2]
        dist_inv = jnp.sqrt(d2) * inv
        t = _dot(u, w1pr[...])
        extra = _dot(jnp.concatenate([dist_inv, rs], axis=1), w1dsr[...])
        h = t * inv + extra + b1r[...]
        h = jnp.maximum(h, 0.0)
        h = jnp.maximum(_dot(h, w2r[...]) + b2r[...], 0.0)
        hx = _dot(h, w3r[...]) + b3r[...]
        out_ref[:, 0:_H] = _ln_from_ext(hx)

    return pl.pallas_call(
        body,
        grid=(nb,),
        in_specs=[
            pl.BlockSpec((_BEB, _W), lambda i: (i, 0)),
            pl.BlockSpec((_BEB, _W), lambda i: (nb + i, 0)),
            _rep((13, _H)), _rep((2, _H)), _rep((1, _H)),
            _rep((_H, _H)), _rep((1, _H)),
            _rep((_H, _W)), _rep((1, _W)),
        ],
        out_specs=pl.BlockSpec((_BEB, _W), lambda i: (i, 0)),
        out_shape=jax.ShapeDtypeStruct((_EH, _W), jnp.float32),
    )(g0, g0, w1p, w1ds, b1, w2, b2.reshape(1, -1), w3x, b3x)


def _edge_pre_tc(edge_latent, w1e, b1):
    nb = _EH // _BEB

    def body(el_ref, w1r, b1r, out_ref):
        out_ref[...] = _dot(el_ref[...][:, 0:_H], w1r[...]) + b1r[...]

    return pl.pallas_call(
        body,
        grid=(nb,),
        in_specs=[
            pl.BlockSpec((_BEB, _W), lambda i: (i, 0)),
            _rep((_H, _H)), _rep((1, _H)),
        ],
        out_specs=pl.BlockSpec((_BEB, _H), lambda i: (i, 0)),
        out_shape=jax.ShapeDtypeStruct((_EH, _H), jnp.float32),
    )(edge_latent, w1e, b1.reshape(1, -1))


def _edge_step_tc(edge_latent, pre, gath, w1sr, w2, b2, w3x, b3x):
    nb = _EH // _BEB

    def body(el_ref, pre_ref, gs_ref, gr_ref, w1srr, w2r, b2r, w3r, b3r,
             out_ref):
        el = el_ref[...][:, 0:_H]
        x = jnp.concatenate(
            [gs_ref[...][:, 0:_H], gr_ref[...][:, 0:_H]], axis=1)
        h = jnp.maximum(pre_ref[...] + _dot(x, w1srr[...]), 0.0)
        h = jnp.maximum(_dot(h, w2r[...]) + b2r[...], 0.0)
        hx = _dot(h, w3r[...]) + b3r[...]
        out_ref[:, 0:_H] = el + _ln_from_ext(hx)

    return pl.pallas_call(
        body,
        grid=(nb,),
        in_specs=[
            pl.BlockSpec((_BEB, _W), lambda i: (i, 0)),
            pl.BlockSpec((_BEB, _H), lambda i: (i, 0)),
            pl.BlockSpec((_BEB, _W), lambda i: (i, 0)),
            pl.BlockSpec((_BEB, _W), lambda i: (nb + i, 0)),
            _rep((2 * _H, _H)),
            _rep((_H, _H)), _rep((1, _H)),
            _rep((_H, _W)), _rep((1, _W)),
        ],
        out_specs=pl.BlockSpec((_BEB, _W), lambda i: (i, 0)),
        out_shape=jax.ShapeDtypeStruct((_EH, _W), jnp.float32),
    )(edge_latent, pre, gath, gath, w1sr, w2, b2.reshape(1, -1),
      w3x, b3x)


def _node_step_tc(node_latent, p1, p2, wb):
    (w1, b1), (w2, b2), (w3, b3) = wb
    nb = _N // _BN

    def body(nl_ref, p1a, p1b, p2a, p2b, w1r, b1r, w2r, b2r, w3r, b3r, out_ref):
        nl = nl_ref[...][:, 0:_H]
        agg = (p1a[...][:, 0:_H] + p1b[...][:, 0:_H]
               + p2a[...][:, 0:_H] + p2b[...][:, 0:_H])
        x = jnp.concatenate([nl, agg], axis=1)
        h = jnp.maximum(_dot(x, w1r[...]) + b1r[...], 0.0)
        h = jnp.maximum(_dot(h, w2r[...]) + b2r[...], 0.0)
        h = _dot(h, w3r[...]) + b3r[...]
        out_ref[:, 0:_H] = nl + _ln(h)

    return pl.pallas_call(
        body,
        grid=(nb,),
        in_specs=[
            pl.BlockSpec((_BN, _W), lambda i: (i, 0)),
            pl.BlockSpec((_BN, _W), lambda i: (i, 0)),
            pl.BlockSpec((_BN, _W), lambda i: (nb + i, 0)),
            pl.BlockSpec((_BN, _W), lambda i: (i, 0)),
            pl.BlockSpec((_BN, _W), lambda i: (nb + i, 0)),
            _rep((2 * _H, _H)), _rep((1, _H)),
            _rep((_H, _H)), _rep((1, _H)),
            _rep((_H, _H)), _rep((1, _H)),
        ],
        out_specs=pl.BlockSpec((_BN, _W), lambda i: (i, 0)),
        out_shape=jax.ShapeDtypeStruct((_N, _W), jnp.float32),
    )(node_latent, p1, p1, p2, p2, w1, b1.reshape(1, -1),
      w2, b2.reshape(1, -1), w3, b3.reshape(1, -1))


def _decoder_tc(node_latent, pos12, wb):
    (w1, b1), (w2, b2), (w3, b3) = wb

    def body(nl_ref, pos_ref, w1r, b1r, w2r, b2r, w3r, b3r, out_ref):
        h = jnp.maximum(_dot(nl_ref[...][:, 0:_H], w1r[...]) + b1r[...], 0.0)
        h = jnp.maximum(_dot(h, w2r[...]) + b2r[...], 0.0)
        acc = _dot(h, w3r[...]) + b3r[...]
        p = pos_ref[...]
        out_ref[...] = 2.0 * p[:, 10:12] - p[:, 8:10] + acc

    return pl.pallas_call(
        body,
        grid=(_N // _BN,),
        in_specs=[
            pl.BlockSpec((_BN, _W), lambda i: (i, 0)),
            pl.BlockSpec((_BN, 12), lambda i: (i, 0)),
            _rep((_H, _H)), _rep((1, _H)),
            _rep((_H, _H)), _rep((1, _H)),
            _rep((_H, 2)), _rep((1, 2)),
        ],
        out_specs=pl.BlockSpec((_BN, 2), lambda i: (i, 0)),
        out_shape=jax.ShapeDtypeStruct((_N, 2), jnp.float32),
    )(node_latent, pos12, w1, b1.reshape(1, -1), w2, b2.reshape(1, -1),
      w3, b3.reshape(1, -1))


def kernel(position_sequence, particle_properties, edge_index, params):
    pos12 = position_sequence.reshape(_N, 12)
    radii = particle_properties
    senders = edge_index[0].astype(jnp.int32)
    receivers = edge_index[1].astype(jnp.int32)
    idx1 = jnp.concatenate([senders[:_EH], receivers[:_EH]])
    idx2 = jnp.concatenate([senders[_EH:], receivers[_EH:]])
    recv1 = receivers[:_EH]
    recv2 = receivers[_EH:]
    zeros_nw = jnp.zeros((_N, _W), jnp.float32)

    (ew1, eb1), (ew2, eb2), (ew3, eb3) = params["edge_enc"]
    w1p = jnp.concatenate(
        [ew1[0:2], ew1[3:13], jnp.zeros((1, _H), ew1.dtype)], axis=0)
    w1ds = jnp.concatenate([ew1[2:3], ew1[13:14]], axis=0)
    ew3x, eb3x = _ext_w3(ew3, eb3)

    table = _encoder_tc(pos12, radii, params["node_enc"])

    g1 = _sc_gather_rows(table, idx1, _E)
    g2 = _sc_gather_rows(table, idx2, _E)
    el1 = _edge_encoder_tc(
        g1, w1p, w1ds, eb1.reshape(1, -1), ew2, eb2, ew3x, eb3x)
    el2 = _edge_encoder_tc(
        g2, w1p, w1ds, eb1.reshape(1, -1), ew2, eb2, ew3x, eb3x)

    node_latent = table
    for s in range(_STEPS):
        (pw1, pb1), (pw2, pb2), (pw3, pb3) = params["proc_edge"][s]
        pw3x, pb3x = _ext_w3(pw3, pb3)
        pre1 = _edge_pre_tc(el1, pw1[0:_H], pb1)
        pre2 = _edge_pre_tc(el2, pw1[0:_H], pb1)
        el1 = _edge_step_tc(el1, pre1, g1, pw1[_H:], pw2, pb2, pw3x, pb3x)
        p1 = _sc_segment_partials(el1, recv1, zeros_nw)
        el2 = _edge_step_tc(el2, pre2, g2, pw1[_H:], pw2, pb2, pw3x, pb3x)
        p2 = _sc_segment_partials(el2, recv2, zeros_nw)
        node_latent = _node_step_tc(node_latent, p1, p2, params["proc_node"][s])
        if s + 1 < _STEPS:
            g1 = _sc_gather_rows(node_latent, idx1, _E)
            g2 = _sc_gather_rows(node_latent, idx2, _E)

    return _decoder_tc(node_latent, pos12, params["decoder"])


# speedup vs baseline: 1.2223x; 1.2223x over previous
import functools

import jax
import jax.numpy as jnp
from jax import lax
from jax.experimental import pallas as pl
from jax.experimental.pallas import tpu as pltpu
from jax.experimental.pallas import tpu_sc as plsc

_N = 10000
_E = 320000
_EH = _E // 2
_H = 64
_W = 128
_STEPS = 4
_NC = 2
_NS = 16
_NW = _NC * _NS
_CH = 128


def _sc_mesh():
    return plsc.VectorSubcoreMesh(
        core_axis_name="c", subcore_axis_name="s",
        num_cores=_NC, num_subcores=_NS)


def _sc_gather_rows(table, idx, n_idx):
    per_w = n_idx // _NW
    n_full = per_w // _CH
    tail = per_w - n_full * _CH
    assert per_w % 8 == 0 and tail % 8 == 0 and n_full % 2 == 0

    @functools.partial(
        pl.kernel,
        out_type=jax.ShapeDtypeStruct((n_idx, _W), jnp.float32),
        mesh=_sc_mesh(),
        scratch_types=[
            pltpu.VMEM((per_w,), jnp.int32),
            pltpu.VMEM((2, _CH, _W), jnp.float32),
            pltpu.SemaphoreType.DMA,
            pltpu.SemaphoreType.DMA,
            pltpu.SemaphoreType.DMA,
            pltpu.SemaphoreType.DMA,
        ],
    )
    def k(table_hbm, idx_hbm, out_hbm, idx_v, rows_v, sg0, sg1, so0, so1):
        wid = lax.axis_index("s") * _NC + lax.axis_index("c")
        base = wid * per_w
        pltpu.sync_copy(idx_hbm.at[pl.ds(base, per_w)], idx_v)
        sg = (sg0, sg1)
        so = (so0, so1)

        def fire_g(i, s):
            pltpu.async_copy(
                table_hbm.at[idx_v.at[pl.ds(i * _CH, _CH)]], rows_v.at[s], sg[s])

        def wait_g(i, s):
            pltpu.make_async_copy(
                table_hbm.at[idx_v.at[pl.ds(i * _CH, _CH)]], rows_v.at[s], sg[s]).wait()

        def fire_o(i, s):
            pltpu.async_copy(
                rows_v.at[s], out_hbm.at[pl.ds(base + i * _CH, _CH)], so[s])

        def wait_o(i, s):
            pltpu.make_async_copy(
                rows_v.at[s], out_hbm.at[pl.ds(base + i * _CH, _CH)], so[s]).wait()

        fire_g(0, 0)

        def body(j, carry):
            for b in range(2):
                i = 2 * j + b
                wait_g(i, b)
                fire_o(i, b)

                @pl.when(i >= 1)
                def _():
                    wait_o(i - 1, 1 - b)

                @pl.when(i + 1 < n_full)
                def _():
                    fire_g(i + 1, 1 - b)
            return carry

        lax.fori_loop(0, n_full // 2, body, 0)
        wait_o(n_full - 1, (n_full - 1) % 2)
        if tail:
            t0 = n_full * _CH
            pltpu.async_copy(
                table_hbm.at[idx_v.at[pl.ds(t0, tail)]],
                rows_v.at[0, pl.ds(0, tail)], sg0).wait()
            pltpu.sync_copy(
                rows_v.at[0, pl.ds(0, tail)],
                out_hbm.at[pl.ds(base + t0, tail)])

    return k(table, idx)


def _sc_segment_partials(values, idx, zeros_nw):
    n_val = values.shape[0]
    per_w = n_val // _NW
    n_full = per_w // _CH
    tail = per_w - n_full * _CH
    assert per_w % 8 == 0 and tail % 8 == 0
    zr0 = 632
    zr_last = _N - zr0 * (_NS - 1)

    @functools.partial(
        pl.kernel,
        out_type=jax.ShapeDtypeStruct((2 * _N, _W), jnp.float32),
        mesh=_sc_mesh(),
        scratch_types=[
            pltpu.VMEM((2, _CH), jnp.int32),
            pltpu.VMEM((tail,), jnp.int32),
            pltpu.VMEM((2, _CH, _W), jnp.float32),
            pltpu.VMEM_SHARED((_N, _W), jnp.float32),
            pltpu.SemaphoreType.DMA,
            pltpu.SemaphoreType.DMA,
        ],
    )
    def k(val_hbm, idx_hbm, zero_hbm, out_hbm, idx_b, idx_t, rows_v, acc, sv0, sv1):
        cid = lax.axis_index("c")
        sid = lax.axis_index("s")
        wid = sid * _NC + cid
        base = wid * per_w

        @pl.when(sid < _NS - 1)
        def _():
            pltpu.sync_copy(zero_hbm.at[pl.ds(sid * zr0, zr0)],
                            acc.at[pl.ds(sid * zr0, zr0)])

        @pl.when(sid == _NS - 1)
        def _():
            pltpu.sync_copy(zero_hbm.at[pl.ds(sid * zr0, zr_last)],
                            acc.at[pl.ds(sid * zr0, zr_last)])

        plsc.subcore_barrier()
        sv = (sv0, sv1)

        def fire_v(i, s):
            pltpu.async_copy(
                val_hbm.at[pl.ds(base + i * _CH, _CH)], rows_v.at[s], sv[s])

        def wait_v(i, s):
            pltpu.make_async_copy(
                val_hbm.at[pl.ds(base + i * _CH, _CH)], rows_v.at[s], sv[s]).wait()

        pltpu.sync_copy(idx_hbm.at[pl.ds(base, _CH)], idx_b.at[0])
        fire_v(0, 0)

        def body(j, carry):
            for b in range(2):
                i = 2 * j + b

                @pl.when(i + 1 < n_full)
                def _():
                    pltpu.sync_copy(
                        idx_hbm.at[pl.ds(base + (i + 1) * _CH, _CH)],
                        idx_b.at[1 - b])
                    fire_v(i + 1, 1 - b)

                wait_v(i, b)
                pltpu.sync_copy(rows_v.at[b], acc.at[idx_b.at[b]], add=True)
            return carry

        lax.fori_loop(0, n_full // 2, body, 0)
        if n_full % 2:
            wait_v(n_full - 1, 0)
            pltpu.sync_copy(rows_v.at[0], acc.at[idx_b.at[0]], add=True)
        if tail:
            t0 = base + n_full * _CH
            pltpu.sync_copy(idx_hbm.at[pl.ds(t0, tail)], idx_t)
            pltpu.async_copy(
                val_hbm.at[pl.ds(t0, tail)],
                rows_v.at[0, pl.ds(0, tail)], sv0).wait()
            pltpu.sync_copy(rows_v.at[0, pl.ds(0, tail)], acc.at[idx_t], add=True)
        plsc.subcore_barrier()

        @pl.when(sid < _NS - 1)
        def _():
            pltpu.sync_copy(acc.at[pl.ds(sid * zr0, zr0)],
                            out_hbm.at[pl.ds(cid * _N + sid * zr0, zr0)])

        @pl.when(sid == _NS - 1)
        def _():
            pltpu.sync_copy(acc.at[pl.ds(sid * zr0, zr_last)],
                            out_hbm.at[pl.ds(cid * _N + sid * zr0, zr_last)])

    return k(values, idx, zeros_nw)


def _ln(h):
    o = jnp.full((_H, 1), 1.0 / _H, jnp.float32)
    mu = jnp.dot(h, o, preferred_element_type=jnp.float32)
    d = h - mu
    var = jnp.dot(d * d, o, preferred_element_type=jnp.float32)
    return d * lax.rsqrt(var + 1e-5)


def _ln_from_ext(hx):
    o = jnp.full((_H, 1), 1.0 / _H, jnp.float32)
    d = hx[:, 0:_H] - hx[:, _H:_H + 1]
    var = jnp.dot(d * d, o, preferred_element_type=jnp.float32)
    return d * lax.rsqrt(var + 1e-5)


def _dot(a, b):
    return jnp.dot(a, b, preferred_element_type=jnp.float32)


def _ext_w3(w3, b3):
    o = jnp.full((_H, 1), 1.0 / _H, jnp.float32)
    w3x = jnp.concatenate(
        [w3, w3 @ o, jnp.zeros((_H, _W - _H - 1), jnp.float32)], axis=1)
    b3r = b3.reshape(1, -1)
    b3x = jnp.concatenate(
        [b3r, b3r @ o, jnp.zeros((1, _W - _H - 1), jnp.float32)], axis=1)
    return w3x, b3x


def _rep(shape):
    return pl.BlockSpec(shape, lambda i: (0, 0))


_BN = 1000
_BEB = 1280


def _encoder_tc(pos12, radii, wb):
    (w1, b1), (w2, b2), (w3, b3) = wb
    grid = (_N // _BN,)

    def body(pos_ref, rad_ref, w1r, b1r, w2r, b2r, w3r, b3r, out_ref):
        p = pos_ref[...]
        r = rad_ref[...]
        vel = p[:, 2:12] - p[:, 0:10]
        cur = p[:, 10:12]
        inv_r = 1.0 / r
        nf = jnp.concatenate(
            [vel * inv_r,
             jnp.clip(jnp.concatenate([cur, 1.0 - cur], axis=1) * inv_r,
                      -4.0, 4.0),
             r], axis=1)
        h = jnp.maximum(_dot(nf, w1r[...]) + b1r[...], 0.0)
        h = jnp.maximum(_dot(h, w2r[...]) + b2r[...], 0.0)
        h = _dot(h, w3r[...]) + b3r[...]
        out_ref[:, 0:_H + 13] = jnp.concatenate([_ln(h), cur, vel, r], axis=1)

    return pl.pallas_call(
        body,
        grid=grid,
        in_specs=[
            pl.BlockSpec((_BN, 12), lambda i: (i, 0)),
            pl.BlockSpec((_BN, 1), lambda i: (i, 0)),
            _rep((15, _H)), _rep((1, _H)),
            _rep((_H, _H)), _rep((1, _H)),
            _rep((_H, _H)), _rep((1, _H)),
        ],
        out_specs=pl.BlockSpec((_BN, _W), lambda i: (i, 0)),
        out_shape=jax.ShapeDtypeStruct((_N, _W), jnp.float32),
    )(pos12, radii, w1, b1.reshape(1, -1), w2, b2.reshape(1, -1),
      w3, b3.reshape(1, -1))


def _edge_encoder_tc(g0, w1p, w1ds, b1, w2, b2, w3x, b3x):
    nb = _EH // _BEB

    def body(s_ref, r_ref, w1pr, w1dsr, b1r, w2r, b2r, w3r, b3r, out_ref):
        s = s_ref[...][:, 64:77]
        r = r_ref[...][:, 64:77]
        u = s - r
        rs = s[:, 12:13] + r[:, 12:13]
        inv = 1.0 / rs
        d2 = u[:, 0:1] * u[:, 0:1] + u[:, 1:2] * u[:, 1:2]
        dist_inv = jnp.sqrt(d2) * inv
        t = _dot(u, w1pr[...])
        extra = _dot(jnp.concatenate([dist_inv, rs], axis=1), w1dsr[...])
        h = t * inv + extra + b1r[...]
        h = jnp.maximum(h, 0.0)
        h = jnp.maximum(_dot(h, w2r[...]) + b2r[...], 0.0)
        hx = _dot(h, w3r[...]) + b3r[...]
        out_ref[:, 0:_H] = _ln_from_ext(hx)

    return pl.pallas_call(
        body,
        grid=(nb,),
        in_specs=[
            pl.BlockSpec((_BEB, _W), lambda i: (i, 0)),
            pl.BlockSpec((_BEB, _W), lambda i: (nb + i, 0)),
            _rep((13, _H)), _rep((2, _H)), _rep((1, _H)),
            _rep((_H, _H)), _rep((1, _H)),
            _rep((_H, _W)), _rep((1, _W)),
        ],
        out_specs=pl.BlockSpec((_BEB, _W), lambda i: (i, 0)),
        out_shape=jax.ShapeDtypeStruct((_EH, _W), jnp.float32),
    )(g0, g0, w1p, w1ds, b1, w2, b2.reshape(1, -1), w3x, b3x)


def _edge_step_tc(edge_latent, gath, w1, b1, w2, b2, w3x, b3x):
    nb = _EH // _BEB

    def body(el_ref, gs_ref, gr_ref, w1r, b1r, w2r, b2r, w3r, b3r, out_ref):
        el = el_ref[...][:, 0:_H]
        x = jnp.concatenate(
            [el, gs_ref[...][:, 0:_H], gr_ref[...][:, 0:_H]], axis=1)
        h = jnp.maximum(_dot(x, w1r[...]) + b1r[...], 0.0)
        h = jnp.maximum(_dot(h, w2r[...]) + b2r[...], 0.0)
        hx = _dot(h, w3r[...]) + b3r[...]
        out_ref[:, 0:_H] = el + _ln_from_ext(hx)

    return pl.pallas_call(
        body,
        grid=(nb,),
        in_specs=[
            pl.BlockSpec((_BEB, _W), lambda i: (i, 0)),
            pl.BlockSpec((_BEB, _W), lambda i: (i, 0)),
            pl.BlockSpec((_BEB, _W), lambda i: (nb + i, 0)),
            _rep((3 * _H, _H)), _rep((1, _H)),
            _rep((_H, _H)), _rep((1, _H)),
            _rep((_H, _W)), _rep((1, _W)),
        ],
        out_specs=pl.BlockSpec((_BEB, _W), lambda i: (i, 0)),
        out_shape=jax.ShapeDtypeStruct((_EH, _W), jnp.float32),
    )(edge_latent, gath, gath, w1, b1.reshape(1, -1), w2, b2.reshape(1, -1),
      w3x, b3x)


def _node_step_tc(node_latent, p1, p2, wb):
    (w1, b1), (w2, b2), (w3, b3) = wb
    nb = _N // _BN

    def body(nl_ref, p1a, p1b, p2a, p2b, w1r, b1r, w2r, b2r, w3r, b3r, out_ref):
        nl = nl_ref[...][:, 0:_H]
        agg = (p1a[...][:, 0:_H] + p1b[...][:, 0:_H]
               + p2a[...][:, 0:_H] + p2b[...][:, 0:_H])
        x = jnp.concatenate([nl, agg], axis=1)
        h = jnp.maximum(_dot(x, w1r[...]) + b1r[...], 0.0)
        h = jnp.maximum(_dot(h, w2r[...]) + b2r[...], 0.0)
        h = _dot(h, w3r[...]) + b3r[...]
        out_ref[:, 0:_H] = nl + _ln(h)

    return pl.pallas_call(
        body,
        grid=(nb,),
        in_specs=[
            pl.BlockSpec((_BN, _W), lambda i: (i, 0)),
            pl.BlockSpec((_BN, _W), lambda i: (i, 0)),
            pl.BlockSpec((_BN, _W), lambda i: (nb + i, 0)),
            pl.BlockSpec((_BN, _W), lambda i: (i, 0)),
            pl.BlockSpec((_BN, _W), lambda i: (nb + i, 0)),
            _rep((2 * _H, _H)), _rep((1, _H)),
            _rep((_H, _H)), _rep((1, _H)),
            _rep((_H, _H)), _rep((1, _H)),
        ],
        out_specs=pl.BlockSpec((_BN, _W), lambda i: (i, 0)),
        out_shape=jax.ShapeDtypeStruct((_N, _W), jnp.float32),
    )(node_latent, p1, p1, p2, p2, w1, b1.reshape(1, -1),
      w2, b2.reshape(1, -1), w3, b3.reshape(1, -1))


def _decoder_tc(node_latent, pos12, wb):
    (w1, b1), (w2, b2), (w3, b3) = wb

    def body(nl_ref, pos_ref, w1r, b1r, w2r, b2r, w3r, b3r, out_ref):
        h = jnp.maximum(_dot(nl_ref[...][:, 0:_H], w1r[...]) + b1r[...], 0.0)
        h = jnp.maximum(_dot(h, w2r[...]) + b2r[...], 0.0)
        acc = _dot(h, w3r[...]) + b3r[...]
        p = pos_ref[...]
        out_ref[...] = 2.0 * p[:, 10:12] - p[:, 8:10] + acc

    return pl.pallas_call(
        body,
        grid=(_N // _BN,),
        in_specs=[
            pl.BlockSpec((_BN, _W), lambda i: (i, 0)),
            pl.BlockSpec((_BN, 12), lambda i: (i, 0)),
            _rep((_H, _H)), _rep((1, _H)),
            _rep((_H, _H)), _rep((1, _H)),
            _rep((_H, 2)), _rep((1, 2)),
        ],
        out_specs=pl.BlockSpec((_BN, 2), lambda i: (i, 0)),
        out_shape=jax.ShapeDtypeStruct((_N, 2), jnp.float32),
    )(node_latent, pos12, w1, b1.reshape(1, -1), w2, b2.reshape(1, -1),
      w3, b3.reshape(1, -1))


def kernel(position_sequence, particle_properties, edge_index, params):
    pos12 = position_sequence.reshape(_N, 12)
    radii = particle_properties
    senders = edge_index[0].astype(jnp.int32)
    receivers = edge_index[1].astype(jnp.int32)
    idx1 = jnp.concatenate([senders[:_EH], receivers[:_EH]])
    idx2 = jnp.concatenate([senders[_EH:], receivers[_EH:]])
    recv1 = receivers[:_EH]
    recv2 = receivers[_EH:]
    zeros_nw = jnp.zeros((_N, _W), jnp.float32)

    (ew1, eb1), (ew2, eb2), (ew3, eb3) = params["edge_enc"]
    w1p = jnp.concatenate(
        [ew1[0:2], ew1[3:13], jnp.zeros((1, _H), ew1.dtype)], axis=0)
    w1ds = jnp.concatenate([ew1[2:3], ew1[13:14]], axis=0)
    ew3x, eb3x = _ext_w3(ew3, eb3)

    table = _encoder_tc(pos12, radii, params["node_enc"])

    g1 = _sc_gather_rows(table, idx1, _E)
    g2 = _sc_gather_rows(table, idx2, _E)
    el1 = _edge_encoder_tc(
        g1, w1p, w1ds, eb1.reshape(1, -1), ew2, eb2, ew3x, eb3x)
    el2 = _edge_encoder_tc(
        g2, w1p, w1ds, eb1.reshape(1, -1), ew2, eb2, ew3x, eb3x)

    node_latent = table
    for s in range(_STEPS):
        (pw1, pb1), (pw2, pb2), (pw3, pb3) = params["proc_edge"][s]
        pw3x, pb3x = _ext_w3(pw3, pb3)
        el1 = _edge_step_tc(el1, g1, pw1, pb1, pw2, pb2, pw3x, pb3x)
        p1 = _sc_segment_partials(el1, recv1, zeros_nw)
        el2 = _edge_step_tc(el2, g2, pw1, pb1, pw2, pb2, pw3x, pb3x)
        p2 = _sc_segment_partials(el2, recv2, zeros_nw)
        node_latent = _node_step_tc(node_latent, p1, p2, params["proc_node"][s])
        if s + 1 < _STEPS:
            g1 = _sc_gather_rows(node_latent, idx1, _E)
            g2 = _sc_gather_rows(node_latent, idx2, _E)

    return _decoder_tc(node_latent, pos12, params["decoder"])
